# baseline (speedup 1.0000x reference)
import jax
import jax.numpy as jnp
from jax import lax
from jax.experimental import pallas as pl
from jax.experimental.pallas import tpu as pltpu
from jax.experimental.pallas import tpu_sc as plsc

_TAU = 0.001
_NBB = 8
_GBLK = 4
_GPW = 16
_CH = 32
_NPN8 = 8


def _sc_body(x_hbm, wn_hbm, o_hbm, p_v, x_v, o_v):
    cid = lax.axis_index("c")
    sid = lax.axis_index("s")
    wid = sid * 2 + cid
    gb = lax.rem(wid, _GBLK) * _GPW
    rpw = x_hbm.shape[0] // _NBB
    rb = (wid // _GBLK) * rpw

    pltpu.sync_copy(wn_hbm.at[:, pl.ds(gb, _GPW)], p_v)

    def smax_g(g, carry):
        for n in range(8):
            vs = [p_v[n, g, pl.ds(i * 16, 16)] for i in range(16)]
            m = vs[0]
            for i in range(1, 16):
                m = jnp.maximum(m, vs[i])
            es = [jnp.exp((vs[i] - m) * (1.0 / _TAU)) for i in range(16)]
            s = es[0]
            for i in range(1, 16):
                s = s + es[i]
            r = 1.0 / s
            for i in range(16):
                p_v[n, g, pl.ds(i * 16, 16)] = es[i] * r
        return carry
    lax.fori_loop(0, _GPW, smax_g, 0)

    def chunk(ch, carry):
        row0 = rb + ch * _CH
        pltpu.sync_copy(x_hbm.at[pl.ds(row0, _CH), pl.ds(gb * 16, _GPW * 16)],
                        x_v)

        def g_body(g, carry2):
            for nq in range(4):
                pv = [[p_v[2 * nq + h, g, pl.ds(i * 16, 16)] for i in range(16)]
                      for h in range(2)]

                def b_body(b, carry3):
                    xv = x_v[b, pl.ds(g * 16, 16)]
                    xs = [xv[i] for i in range(16)]
                    for h in range(2):
                        acc = pv[h][0] * xs[0]
                        for i in range(1, 16):
                            acc = acc + pv[h][i] * xs[i]
                        o_v[2 * nq + h, b, pl.ds(g * 16, 16)] = acc
                    return carry3
                lax.fori_loop(0, _CH, b_body, 0)
            return carry2
        lax.fori_loop(0, _GPW, g_body, 0)

        for n in range(8):
            pltpu.sync_copy(
                o_v.at[n],
                o_hbm.at[pl.ds(row0, _CH), n, pl.ds(gb * 16, _GPW * 16)])
        return carry
    lax.fori_loop(0, x_hbm.shape[0] // _NBB // _CH, chunk, 0)


def kernel(x, W):
    B = x.shape[0]
    wn = jnp.transpose(W, (2, 0, 3, 1)).reshape(_NPN8, 64, 256)
    mesh = plsc.VectorSubcoreMesh(core_axis_name="c", subcore_axis_name="s",
                                  num_cores=2, num_subcores=16)
    out3 = pl.kernel(
        _sc_body,
        out_type=jax.ShapeDtypeStruct((B, 8, 1024), jnp.float32),
        mesh=mesh,
        scratch_types=[
            pltpu.VMEM((8, _GPW, 256), jnp.float32),
            pltpu.VMEM((_CH, _GPW * 16), jnp.float32),
            pltpu.VMEM((8, _CH, _GPW * 16), jnp.float32),
        ],
    )(x, wn)
    return jnp.transpose(out3, (0, 2, 1))


# speedup vs baseline: 15.2551x; 15.2551x over previous
import jax
import jax.numpy as jnp
from jax.experimental import pallas as pl
from jax.experimental.pallas import tpu as pltpu

_TAU = 0.001
_G = 64
_GS = 16
_NG = 16
_NPN = 8
_NC = 8
_BB = 512


def _copy(buf_ref, o_ref, sem, slot, n, step):
    return pltpu.make_async_copy(
        buf_ref.at[slot, n],
        o_ref.at[pl.ds(step * _BB, _BB), n, :],
        sem.at[slot, n])


def _fwd_kernel(x_ref, w_ref, o_ref, buf_ref, m_ref, sem):
    i = pl.program_id(0)
    nsteps = pl.num_programs(0)
    slot = jax.lax.rem(i, 2)

    @pl.when(i == 0)
    def _():
        logits = w_ref[...] * (1.0 / _TAU)
        mx = jnp.max(logits, axis=2, keepdims=True)
        e = jnp.exp(logits - mx)
        p = (e / jnp.sum(e, axis=2, keepdims=True)).astype(jnp.bfloat16)
        rows = jax.lax.broadcasted_iota(jnp.int32, (128, 128), 0)
        cols = jax.lax.broadcasted_iota(jnp.int32, (128, 128), 1)
        mask = (rows // _GS) == (cols // _GS)
        zero = jnp.zeros((128, 128), jnp.bfloat16)
        for c in range(_NC):
            for n in range(_NPN):
                s = p[n, 8 * c:8 * c + 8].reshape(128, _GS)
                t = jnp.tile(s, (1, 8))
                bd = jnp.where(mask, t, zero)
                m_ref[c, n // 2, :, (n % 2) * 128:(n % 2) * 128 + 128] = bd

    @pl.when(i >= 2)
    def _():
        for n in range(_NPN):
            _copy(buf_ref, o_ref, sem, slot, n, i - 2).wait()

    x = x_ref[...].astype(jnp.bfloat16)
    for c in range(_NC):
        xc = x[:, c * 128:(c + 1) * 128]
        for k in range(_NPN // 2):
            r = jnp.dot(xc, m_ref[c, k], preferred_element_type=jnp.float32)
            buf_ref[slot, 2 * k, :, c * 128:(c + 1) * 128] = r[:, :128]
            buf_ref[slot, 2 * k + 1, :, c * 128:(c + 1) * 128] = r[:, 128:]

    for n in range(_NPN):
        _copy(buf_ref, o_ref, sem, slot, n, i).start()

    @pl.when(i == nsteps - 1)
    def _():
        for n in range(_NPN):
            _copy(buf_ref, o_ref, sem, 1 - slot, n, i - 1).wait()
        for n in range(_NPN):
            _copy(buf_ref, o_ref, sem, slot, n, i).wait()


def kernel(x, W):
    B = x.shape[0]
    wn = jnp.transpose(W, (2, 0, 3, 1))
    out3 = pl.pallas_call(
        _fwd_kernel,
        grid=(B // _BB,),
        in_specs=[
            pl.BlockSpec((_BB, _G * _GS), lambda i: (i, 0)),
            pl.BlockSpec((_NPN, _G, _GS, _NG), lambda i: (0, 0, 0, 0)),
        ],
        out_specs=pl.BlockSpec(memory_space=pltpu.MemorySpace.HBM),
        out_shape=jax.ShapeDtypeStruct((B, _NPN, 1024), jnp.float32),
        scratch_shapes=[
            pltpu.VMEM((2, _NPN, _BB, 1024), jnp.float32),
            pltpu.VMEM((_NC, _NPN // 2, 128, 256), jnp.bfloat16),
            pltpu.SemaphoreType.DMA((2, _NPN)),
        ],
    )(x, wn)
    return jnp.transpose(out3, (0, 2, 1))
